# baseline (speedup 1.0000x reference)
import functools

import jax
import jax.numpy as jnp
from jax import lax
from jax.experimental import pallas as pl
from jax.experimental.pallas import tpu as pltpu
from jax.experimental.pallas import tpu_sc as plsc

B = 16384
N_SPARSE = 26
N_DENSE = 13
N_FIELDS = N_SPARSE + N_DENSE
N_PAD = 48
VOCAB = 1000000

NC = 2
NS = 16
NW = NC * NS
ROWS_W = B // NW
CHUNK = 128
NCHUNK = ROWS_W // CHUNK
BCOLS = B // CHUNK


def _body(xi_hbm, xd_hbm, tab_hbm, ap_hbm, wp_hbm, out_hbm,
          idx_v, gat_v, xd_v, coef_v, cb_v, acc_v, ap_v, wp_v, sem):
    w = lax.axis_index("s") * NC + lax.axis_index("c")
    col0 = w * NCHUNK

    pltpu.sync_copy(xi_hbm.at[:, pl.ds(col0, NCHUNK), :], idx_v)
    pltpu.sync_copy(xd_hbm.at[:, pl.ds(col0, NCHUNK), :], xd_v)
    pltpu.sync_copy(ap_hbm, ap_v)
    pltpu.sync_copy(wp_hbm, wp_v)

    for i in range(N_PAD // 16):
        sl = pl.ds(i * 16, 16)
        e = jnp.exp(2.0 * ap_v[sl])
        coef_v[sl] = (e - 1.0) / (e + 1.0) * wp_v[sl]
    for f in range(N_FIELDS):
        grp = coef_v[pl.ds((f // 16) * 16, 16)]
        lane = jnp.full((16,), f % 16, dtype=jnp.int32)
        cb_v[pl.ds(f * 16, 16)] = grp.at[lane].get(mode="promise_in_bounds")

    def add_off(t, carry):
        f = t // (NCHUNK * (CHUNK // 16))
        r = (t // (CHUNK // 16)) % NCHUNK
        i = t % (CHUNK // 16)
        sl = pl.ds(i * 16, 16)
        idx_v[f, r, sl] = idx_v[f, r, sl] + f * VOCAB
        return carry
    lax.fori_loop(0, N_SPARSE * NCHUNK * (CHUNK // 16), add_off, 0)

    copies = []
    for f in range(N_SPARSE):
        for r in range(NCHUNK):
            cp = pltpu.make_async_copy(
                tab_hbm.at[idx_v.at[f, r]], gat_v.at[f, r], sem)
            cp.start()
            copies.append(cp)
    for cp in copies:
        cp.wait()

    def reduce_chunk(t, carry):
        r = t // (CHUNK // 16)
        i = t % (CHUNK // 16)
        sl = pl.ds(i * 16, 16)
        acc = jnp.zeros((16,), dtype=jnp.float32)
        for f in range(N_SPARSE):
            acc = acc + cb_v[pl.ds(f * 16, 16)] * gat_v[f, r, sl]
        for d in range(N_DENSE):
            acc = acc + cb_v[pl.ds((N_SPARSE + d) * 16, 16)] * xd_v[d, r, sl]
        acc_v[r, sl] = acc
        return carry
    lax.fori_loop(0, NCHUNK * (CHUNK // 16), reduce_chunk, 0)

    pltpu.sync_copy(acc_v, out_hbm.at[pl.ds(col0, NCHUNK), :])


@jax.jit
def _run(xi3, xd3, tab, apad, wpad):
    mesh = plsc.VectorSubcoreMesh(core_axis_name="c", subcore_axis_name="s")
    fn = pl.kernel(
        _body,
        out_type=jax.ShapeDtypeStruct((BCOLS, CHUNK), jnp.float32),
        mesh=mesh,
        scratch_types=[
            pltpu.VMEM((N_SPARSE, NCHUNK, CHUNK), jnp.int32),
            pltpu.VMEM((N_SPARSE, NCHUNK, CHUNK), jnp.float32),
            pltpu.VMEM((N_DENSE, NCHUNK, CHUNK), jnp.float32),
            pltpu.VMEM((N_PAD,), jnp.float32),
            pltpu.VMEM((N_FIELDS * 16,), jnp.float32),
            pltpu.VMEM((NCHUNK, CHUNK), jnp.float32),
            pltpu.VMEM((N_PAD,), jnp.float32),
            pltpu.VMEM((N_PAD,), jnp.float32),
            pltpu.SemaphoreType.DMA,
        ],
    )
    return fn(xi3, xd3, tab, apad, wpad)


def kernel(X, tables, weight, alpha):
    xi3 = X[:, :N_SPARSE].astype(jnp.int32).T.reshape(N_SPARSE, BCOLS, CHUNK)
    xd3 = X[:, N_SPARSE:].T.reshape(N_DENSE, BCOLS, CHUNK)
    tab = tables.reshape(-1)
    apad = jnp.pad(alpha, (0, N_PAD - N_FIELDS))
    wpad = jnp.concatenate(
        [jnp.ones((N_SPARSE,), jnp.float32), weight[:, 0],
         jnp.zeros((N_PAD - N_FIELDS,), jnp.float32)])
    out = _run(xi3, xd3, tab, apad, wpad)
    return out.reshape(B, 1)
